# baseline (speedup 1.0000x reference)
import functools

import jax
import jax.numpy as jnp
from jax import lax
from jax.experimental import pallas as pl
from jax.experimental.pallas import tpu as pltpu
from jax.experimental.pallas import tpu_sc as plsc

_N = 50000
_E = 800000
_D = 32
_R = 4
_H1 = 32
_H2 = 16

_NC, _NS = 2, 16
_NW = _NC * _NS
_ROWS = 2 * _E
_PER_W = _ROWS // _NW
_CHUNK = 2000
_NITER = _PER_W // _CHUNK

_B = 3200


def _sc_gather(x, flat_idx):
    mesh = plsc.VectorSubcoreMesh(core_axis_name="c", subcore_axis_name="s")

    @functools.partial(
        pl.kernel,
        mesh=mesh,
        compiler_params=pltpu.CompilerParams(use_tc_tiling_on_sc=False),
        out_type=jax.ShapeDtypeStruct((_ROWS, _D), jnp.float32),
        scratch_types=[
            pltpu.VMEM((_CHUNK,), jnp.int32),
            pltpu.VMEM((_CHUNK, _D), jnp.float32),
            pltpu.SemaphoreType.DMA,
        ],
    )
    def gather_k(x_hbm, idx_hbm, out_hbm, idx_v, rows_v, sem):
        wid = lax.axis_index("s") * _NC + lax.axis_index("c")
        base = wid * _PER_W

        def body(i, carry):
            off = base + i * _CHUNK
            pltpu.sync_copy(idx_hbm.at[pl.ds(off, _CHUNK)], idx_v)
            pltpu.async_copy(x_hbm.at[idx_v], rows_v, sem).wait()
            pltpu.sync_copy(rows_v, out_hbm.at[pl.ds(off, _CHUNK)])
            return carry

        lax.fori_loop(0, _NITER, body, 0)

    return gather_k(x, flat_idx)


def _tc_body(g_ref, c_ref, m_ref, r_ref, w1a_ref, w1b_ref, wca_ref, wcb_ref,
             b1_ref, w2_ref, b2_ref, w3_ref, b3_ref, o_ref):
    g = g_ref[...]
    gr = g[:, :_D]
    gc = g[:, _D:]
    m = m_ref[...]
    ga = gr * m + gc * (1.0 - m)
    gb = gc * m + gr * (1.0 - m)
    c = c_ref[...]
    c0 = c[:, 0:1]
    c1 = c[:, 1:2]
    ca = c0 * m + c1 * (1.0 - m)
    cb = c1 * m + c0 * (1.0 - m)
    h = jnp.dot(ga, w1a_ref[...], preferred_element_type=jnp.float32)
    h = h + jnp.dot(gb, w1b_ref[...], preferred_element_type=jnp.float32)
    h = h + ca * wca_ref[...] + cb * wcb_ref[...] + b1_ref[...]
    rel = r_ref[...]
    lane = lax.broadcasted_iota(jnp.int32, (_B, _R * _H1), 1)
    hm = jnp.where((lane // _H1) == rel, h, 0.0)
    h1 = (hm[:, 0:32] + hm[:, 32:64] + hm[:, 64:96] + hm[:, 96:128])
    h1 = jnp.maximum(h1, 0.0)
    h2 = jnp.dot(h1, w2_ref[...], preferred_element_type=jnp.float32)
    h2 = jnp.maximum(h2 + b2_ref[...], 0.0)
    o = jnp.dot(h2, w3_ref[...], preferred_element_type=jnp.float32)
    o_ref[...] = o + b3_ref[...]


def _tc_mlp(g2, concs, maskf, rel, w1a, w1b, wca, wcb, b1c, W2, b2, W3, b3):
    grid = (_E // _B,)
    edge_spec = lambda d: pl.BlockSpec((_B, d), lambda i: (i, 0))
    full = lambda s: pl.BlockSpec(s, lambda i: (0, 0))
    return pl.pallas_call(
        _tc_body,
        grid=grid,
        in_specs=[
            edge_spec(2 * _D),
            edge_spec(2),
            edge_spec(1),
            edge_spec(1),
            full((_D, _R * _H1)),
            full((_D, _R * _H1)),
            full((1, _R * _H1)),
            full((1, _R * _H1)),
            full((1, _R * _H1)),
            full((_H1, _H2)),
            full((1, _H2)),
            full((_H2, 1)),
            full((1, 1)),
        ],
        out_specs=edge_spec(1),
        out_shape=jax.ShapeDtypeStruct((_E, 1), jnp.float32),
    )(g2, concs, maskf, rel, w1a, w1b, wca, wcb, b1c, W2, b2, W3, b3)


def kernel(edge_index, relations, concs, x, W1, b1, W2, b2, W3, b3):
    flat_idx = edge_index.reshape(_ROWS).astype(jnp.int32)
    g = _sc_gather(x, flat_idx)
    g2 = g.reshape(_E, 2 * _D)

    maskf = (jax.random.uniform(jax.random.key(42), (_E,)) >= 0.5)
    maskf = maskf.astype(jnp.float32).reshape(_E, 1)
    rel = relations.astype(jnp.int32).reshape(_E, 1)

    w1cat = W1.transpose(1, 0, 2).reshape(2 * (_D + 1), _R * _H1)
    w1a = w1cat[0:_D]
    wca = w1cat[_D:_D + 1]
    w1b = w1cat[_D + 1:2 * _D + 1]
    wcb = w1cat[2 * _D + 1:2 * _D + 2]
    b1c = b1.reshape(1, _R * _H1)

    return _tc_mlp(g2, concs, maskf, rel, w1a, w1b, wca, wcb, b1c,
                   W2, b2.reshape(1, _H2), W3, b3.reshape(1, 1))


# speedup vs baseline: 3.0796x; 3.0796x over previous
import functools

import jax
import jax.numpy as jnp
from jax import lax
from jax.experimental import pallas as pl
from jax.experimental.pallas import tpu as pltpu
from jax.experimental.pallas import tpu_sc as plsc

_N = 50000
_E = 800000
_D = 32
_R = 4
_H1 = 32
_H2 = 16

_NC, _NS = 2, 16
_NW = _NC * _NS
_PER_W = _E // _NW
_CHUNK = 1000
_NITER = _PER_W // _CHUNK

_B = 3200


def _sc_gather(x, idx_r, idx_c):
    mesh = plsc.VectorSubcoreMesh(core_axis_name="c", subcore_axis_name="s")

    @functools.partial(
        pl.kernel,
        mesh=mesh,
        compiler_params=pltpu.CompilerParams(use_tc_tiling_on_sc=False),
        out_type=(jax.ShapeDtypeStruct((_E, _D), jnp.float32),
                  jax.ShapeDtypeStruct((_E, _D), jnp.float32)),
        scratch_types=[
            pltpu.VMEM((_CHUNK,), jnp.int32),
            pltpu.VMEM((_CHUNK,), jnp.int32),
            pltpu.VMEM((_CHUNK, _D), jnp.float32),
            pltpu.VMEM((_CHUNK, _D), jnp.float32),
            pltpu.SemaphoreType.DMA,
            pltpu.SemaphoreType.DMA,
        ],
    )
    def gather_k(x_hbm, idxr_hbm, idxc_hbm, outr_hbm, outc_hbm,
                 idxr_v, idxc_v, rowsr_v, rowsc_v, semr, semc):
        wid = lax.axis_index("s") * _NC + lax.axis_index("c")
        base = wid * _PER_W

        def body(i, carry):
            off = base + i * _CHUNK
            pltpu.sync_copy(idxr_hbm.at[pl.ds(off, _CHUNK)], idxr_v)
            pltpu.sync_copy(idxc_hbm.at[pl.ds(off, _CHUNK)], idxc_v)
            cr = pltpu.async_copy(x_hbm.at[idxr_v], rowsr_v, semr)
            cc = pltpu.async_copy(x_hbm.at[idxc_v], rowsc_v, semc)
            cr.wait()
            cc.wait()
            pltpu.sync_copy(rowsr_v, outr_hbm.at[pl.ds(off, _CHUNK)])
            pltpu.sync_copy(rowsc_v, outc_hbm.at[pl.ds(off, _CHUNK)])
            return carry

        lax.fori_loop(0, _NITER, body, 0)

    return gather_k(x, idx_r, idx_c)


def _tc_body(gr_ref, gc_ref, s_ref, ws_ref, wd_ref, vs_ref, vd_ref, ug_ref,
             ur_ref, b1_ref, w2t_ref, b2_ref, w3_ref, b3_ref, o_ref):
    gr = gr_ref[...]
    gc = gc_ref[...]
    s = s_ref[...]
    u = (jnp.dot(gr + gc, ws_ref[...], preferred_element_type=jnp.float32)
         + jnp.dot(s, vs_ref[...], preferred_element_type=jnp.float32))
    v = (jnp.dot(gr - gc, wd_ref[...], preferred_element_type=jnp.float32)
         + jnp.dot(s, vd_ref[...], preferred_element_type=jnp.float32))
    sg = jnp.dot(s, ug_ref[...], preferred_element_type=jnp.float32)
    r128 = jnp.dot(s, ur_ref[...], preferred_element_type=jnp.float32)
    h = (u - v) + sg * v + b1_ref[...]
    lane = lax.broadcasted_iota(jnp.int32, (_B, _R * _H1), 1)
    group = (lane // _H1).astype(jnp.float32)
    hm = jnp.where(group == r128, jnp.maximum(h, 0.0), 0.0)
    h2 = jnp.dot(hm, w2t_ref[...], preferred_element_type=jnp.float32)
    h2 = jnp.maximum(h2 + b2_ref[...], 0.0)
    o = jnp.dot(h2, w3_ref[...], preferred_element_type=jnp.float32)
    o_ref[...] = o + b3_ref[...]


def _tc_mlp(gr, gc, s, ws, wd, vs, vd, ug, ur, b1c, w2t, b2, W3, b3):
    grid = (_E // _B,)
    edge_spec = lambda d: pl.BlockSpec((_B, d), lambda i: (i, 0))
    full = lambda sh: pl.BlockSpec(sh, lambda i: (0, 0))
    return pl.pallas_call(
        _tc_body,
        grid=grid,
        in_specs=[
            edge_spec(_D),
            edge_spec(_D),
            edge_spec(4),
            full((_D, _R * _H1)),
            full((_D, _R * _H1)),
            full((4, _R * _H1)),
            full((4, _R * _H1)),
            full((4, _R * _H1)),
            full((4, _R * _H1)),
            full((1, _R * _H1)),
            full((_R * _H1, _H2)),
            full((1, _H2)),
            full((_H2, 1)),
            full((1, 1)),
        ],
        out_specs=edge_spec(1),
        out_shape=jax.ShapeDtypeStruct((_E, 1), jnp.float32),
    )(gr, gc, s, ws, wd, vs, vd, ug, ur, b1c, w2t, b2, W3, b3)


def kernel(edge_index, relations, concs, x, W1, b1, W2, b2, W3, b3):
    idx_r = edge_index[:, 0].astype(jnp.int32)
    idx_c = edge_index[:, 1].astype(jnp.int32)
    gr, gc = _sc_gather(x, idx_r, idx_c)

    maskf = (jax.random.uniform(jax.random.key(42), (_E,)) >= 0.5)
    s = jnp.concatenate(
        [maskf.astype(jnp.float32)[:, None], concs,
         relations.astype(jnp.float32)[:, None]], axis=1)

    w1cat = W1.transpose(1, 0, 2).reshape(2 * (_D + 1), _R * _H1)
    w1a = w1cat[0:_D]
    wca = w1cat[_D:_D + 1]
    w1b = w1cat[_D + 1:2 * _D + 1]
    wcb = w1cat[2 * _D + 1:2 * _D + 2]

    kdim = _R * _H1
    zrow = jnp.zeros((1, kdim), jnp.float32)
    ones = jnp.ones((1, kdim), jnp.float32)
    ws = (w1a + w1b) * 0.5
    wd = (w1a - w1b) * 0.5
    vcs = (wca + wcb) * 0.5
    vcd = (wca - wcb) * 0.5
    vs = jnp.concatenate([zrow, vcs, vcs, zrow], axis=0)
    vd = jnp.concatenate([zrow, vcd, -vcd, zrow], axis=0)
    ug = jnp.concatenate([2.0 * ones, zrow, zrow, zrow], axis=0)
    ur = jnp.concatenate([zrow, zrow, zrow, ones], axis=0)
    w2t = jnp.tile(W2, (_R, 1))

    return _tc_mlp(gr, gc, s, ws, wd, vs, vd, ug, ur, b1.reshape(1, kdim),
                   w2t, b2.reshape(1, _H2), W3, b3.reshape(1, 1))


# speedup vs baseline: 3.1344x; 1.0178x over previous
import functools

import jax
import jax.numpy as jnp
from jax import lax
from jax.experimental import pallas as pl
from jax.experimental.pallas import tpu as pltpu
from jax.experimental.pallas import tpu_sc as plsc

_N = 50000
_E = 800000
_D = 32
_R = 4
_H1 = 32
_H2 = 16

_NC, _NS = 2, 16
_NW = _NC * _NS
_PER_W = _E // _NW
_CHUNK = 1000
_NITER = _PER_W // _CHUNK

_B = 3200


def _sc_gather(x, idx_r, idx_c):
    mesh = plsc.VectorSubcoreMesh(core_axis_name="c", subcore_axis_name="s")

    @functools.partial(
        pl.kernel,
        mesh=mesh,
        compiler_params=pltpu.CompilerParams(use_tc_tiling_on_sc=False),
        out_type=(jax.ShapeDtypeStruct((_E, _D), jnp.float32),
                  jax.ShapeDtypeStruct((_E, _D), jnp.float32)),
        scratch_types=[
            pltpu.VMEM((_CHUNK,), jnp.int32),
            pltpu.VMEM((_CHUNK,), jnp.int32),
            pltpu.VMEM((_CHUNK, _D), jnp.float32),
            pltpu.VMEM((_CHUNK, _D), jnp.float32),
            pltpu.SemaphoreType.DMA,
            pltpu.SemaphoreType.DMA,
        ],
    )
    def gather_k(x_hbm, idxr_hbm, idxc_hbm, outr_hbm, outc_hbm,
                 idxr_v, idxc_v, rowsr_v, rowsc_v, semr, semc):
        wid = lax.axis_index("s") * _NC + lax.axis_index("c")
        base = wid * _PER_W

        def body(i, carry):
            off = base + i * _CHUNK
            pltpu.sync_copy(idxr_hbm.at[pl.ds(off, _CHUNK)], idxr_v)
            pltpu.sync_copy(idxc_hbm.at[pl.ds(off, _CHUNK)], idxc_v)
            cr = pltpu.async_copy(x_hbm.at[idxr_v], rowsr_v, semr)
            cc = pltpu.async_copy(x_hbm.at[idxc_v], rowsc_v, semc)
            cr.wait()
            cc.wait()
            pltpu.sync_copy(rowsr_v, outr_hbm.at[pl.ds(off, _CHUNK)])
            pltpu.sync_copy(rowsc_v, outc_hbm.at[pl.ds(off, _CHUNK)])
            return carry

        lax.fori_loop(0, _NITER, body, 0)

    return gather_k(x, idx_r, idx_c)


def _tc_body(gr_ref, gc_ref, s_ref, ws_ref, wd_ref, vs_ref, vd_ref, ug_ref,
             ur_ref, b1_ref, w2t_ref, b2_ref, w3_ref, b3_ref, o_ref):
    gr = gr_ref[...]
    gc = gc_ref[...]
    s = s_ref[...]
    u = (jnp.dot(gr + gc, ws_ref[...], preferred_element_type=jnp.float32)
         + jnp.dot(s, vs_ref[...], preferred_element_type=jnp.float32))
    v = (jnp.dot(gr - gc, wd_ref[...], preferred_element_type=jnp.float32)
         + jnp.dot(s, vd_ref[...], preferred_element_type=jnp.float32))
    sg = jnp.dot(s, ug_ref[...], preferred_element_type=jnp.float32)
    r128 = jnp.dot(s, ur_ref[...], preferred_element_type=jnp.float32)
    h = (u - v) + sg * v + b1_ref[...]
    lane = lax.broadcasted_iota(jnp.int32, (_B, _R * _H1), 1)
    group = (lane // _H1).astype(jnp.float32)
    hm = jnp.where(group == r128, jnp.maximum(h, 0.0), 0.0)
    h2 = jnp.dot(hm, w2t_ref[...], preferred_element_type=jnp.float32)
    h2 = jnp.maximum(h2 + b2_ref[...], 0.0)
    o = jnp.dot(h2, w3_ref[...], preferred_element_type=jnp.float32)
    o_ref[...] = o + b3_ref[...]


def _tc_mlp(gr, gc, s, ws, wd, vs, vd, ug, ur, b1c, w2t, b2, W3, b3):
    grid = (_E // _B,)
    edge_spec = lambda d: pl.BlockSpec((_B, d), lambda i: (i, 0))
    full = lambda sh: pl.BlockSpec(sh, lambda i: (0, 0))
    return pl.pallas_call(
        _tc_body,
        grid=grid,
        in_specs=[
            edge_spec(_D),
            edge_spec(_D),
            edge_spec(4),
            full((_D, _R * _H1)),
            full((_D, _R * _H1)),
            full((4, _R * _H1)),
            full((4, _R * _H1)),
            full((4, _R * _H1)),
            full((4, _R * _H1)),
            full((1, _R * _H1)),
            full((_R * _H1, _H2)),
            full((1, _H2)),
            full((_H2, 1)),
            full((1, 1)),
        ],
        out_specs=edge_spec(1),
        out_shape=jax.ShapeDtypeStruct((_E, 1), jnp.float32),
    )(gr, gc, s, ws, wd, vs, vd, ug, ur, b1c, w2t, b2, W3, b3)


def kernel(edge_index, relations, concs, x, W1, b1, W2, b2, W3, b3):
    idx_r = edge_index[:, 0].astype(jnp.int32)
    idx_c = edge_index[:, 1].astype(jnp.int32)
    gr, gc = _sc_gather(x, idx_r, idx_c)

    with jax.ensure_compile_time_eval():
        maskf = (jax.random.uniform(jax.random.key(42), (_E,))
                 >= 0.5).astype(jnp.float32)[:, None]
    s = jnp.concatenate(
        [maskf, concs, relations.astype(jnp.float32)[:, None]], axis=1)

    w1cat = W1.transpose(1, 0, 2).reshape(2 * (_D + 1), _R * _H1)
    w1a = w1cat[0:_D]
    wca = w1cat[_D:_D + 1]
    w1b = w1cat[_D + 1:2 * _D + 1]
    wcb = w1cat[2 * _D + 1:2 * _D + 2]

    kdim = _R * _H1
    zrow = jnp.zeros((1, kdim), jnp.float32)
    ones = jnp.ones((1, kdim), jnp.float32)
    ws = (w1a + w1b) * 0.5
    wd = (w1a - w1b) * 0.5
    vcs = (wca + wcb) * 0.5
    vcd = (wca - wcb) * 0.5
    vs = jnp.concatenate([zrow, vcs, vcs, zrow], axis=0)
    vd = jnp.concatenate([zrow, vcd, -vcd, zrow], axis=0)
    ug = jnp.concatenate([2.0 * ones, zrow, zrow, zrow], axis=0)
    ur = jnp.concatenate([zrow, zrow, zrow, ones], axis=0)
    w2t = jnp.tile(W2, (_R, 1))

    return _tc_mlp(gr, gc, s, ws, wd, vs, vd, ug, ur, b1.reshape(1, kdim),
                   w2t, b2.reshape(1, _H2), W3, b3.reshape(1, 1))


# speedup vs baseline: 3.5409x; 1.1297x over previous
import functools

import jax
import jax.numpy as jnp
from jax import lax
from jax.experimental import pallas as pl
from jax.experimental.pallas import tpu as pltpu
from jax.experimental.pallas import tpu_sc as plsc

_N = 50000
_E = 800000
_D = 32
_R = 4
_H1 = 32
_H2 = 16

_NC, _NS = 2, 16
_NW = _NC * _NS
_PER_W = _E // _NW
_CHUNK = 1000
_NITER = _PER_W // _CHUNK

_B = 3200


def _sc_gather(x, idx2):
    mesh = plsc.VectorSubcoreMesh(core_axis_name="c", subcore_axis_name="s")

    @functools.partial(
        pl.kernel,
        mesh=mesh,
        compiler_params=pltpu.CompilerParams(use_tc_tiling_on_sc=False),
        out_type=(jax.ShapeDtypeStruct((_E, _D), jnp.float32),
                  jax.ShapeDtypeStruct((_E, _D), jnp.float32)),
        scratch_types=[
            pltpu.VMEM((_CHUNK,), jnp.int32),
            pltpu.VMEM((_CHUNK,), jnp.int32),
            pltpu.VMEM((_CHUNK, _D), jnp.float32),
            pltpu.VMEM((_CHUNK, _D), jnp.float32),
            pltpu.SemaphoreType.DMA,
            pltpu.SemaphoreType.DMA,
        ],
    )
    def gather_k(x_hbm, idx2_hbm, outr_hbm, outc_hbm,
                 idxr_v, idxc_v, rowsr_v, rowsc_v, semr, semc):
        wid = lax.axis_index("s") * _NC + lax.axis_index("c")
        base = wid * _PER_W

        def body(i, carry):
            off = base + i * _CHUNK
            pltpu.sync_copy(idx2_hbm.at[0, pl.ds(off, _CHUNK)], idxr_v)
            pltpu.sync_copy(idx2_hbm.at[1, pl.ds(off, _CHUNK)], idxc_v)
            cr = pltpu.async_copy(x_hbm.at[idxr_v], rowsr_v, semr)
            cc = pltpu.async_copy(x_hbm.at[idxc_v], rowsc_v, semc)
            cr.wait()
            cc.wait()
            pltpu.sync_copy(rowsr_v, outr_hbm.at[pl.ds(off, _CHUNK)])
            pltpu.sync_copy(rowsc_v, outc_hbm.at[pl.ds(off, _CHUNK)])
            return carry

        lax.fori_loop(0, _NITER, body, 0)

    return gather_k(x, idx2)


def _sdot(st, v):
    return lax.dot_general(st, v, (((0,), (0,)), ((), ())),
                           preferred_element_type=jnp.float32)


def _tc_body(gr_ref, gc_ref, s_ref, ws_ref, wd_ref, vs_ref, vd_ref, ug_ref,
             ur_ref, b1_ref, w2t_ref, b2_ref, w3_ref, b3_ref, o_ref):
    gr = gr_ref[...]
    gc = gc_ref[...]
    st = s_ref[...]
    u = (jnp.dot(gr + gc, ws_ref[...], preferred_element_type=jnp.float32)
         + _sdot(st, vs_ref[...]))
    v = (jnp.dot(gr - gc, wd_ref[...], preferred_element_type=jnp.float32)
         + _sdot(st, vd_ref[...]))
    sg = _sdot(st, ug_ref[...])
    r128 = _sdot(st, ur_ref[...])
    h = (u - v) + sg * v + b1_ref[...]
    lane = lax.broadcasted_iota(jnp.int32, (_B, _R * _H1), 1)
    group = (lane // _H1).astype(jnp.float32)
    hm = jnp.where(group == r128, jnp.maximum(h, 0.0), 0.0)
    h2 = jnp.dot(hm, w2t_ref[...], preferred_element_type=jnp.float32)
    h2 = jnp.maximum(h2 + b2_ref[...], 0.0)
    o = jnp.dot(h2, w3_ref[...], preferred_element_type=jnp.float32)
    o_ref[...] = o + b3_ref[...]


def _tc_mlp(gr, gc, s, ws, wd, vs, vd, ug, ur, b1c, w2t, b2, W3, b3):
    grid = (_E // _B,)
    edge_spec = lambda d: pl.BlockSpec((_B, d), lambda i: (i, 0))
    full = lambda sh: pl.BlockSpec(sh, lambda i: (0, 0))
    return pl.pallas_call(
        _tc_body,
        grid=grid,
        in_specs=[
            edge_spec(_D),
            edge_spec(_D),
            pl.BlockSpec((4, _B), lambda i: (0, i)),
            full((_D, _R * _H1)),
            full((_D, _R * _H1)),
            full((4, _R * _H1)),
            full((4, _R * _H1)),
            full((4, _R * _H1)),
            full((4, _R * _H1)),
            full((1, _R * _H1)),
            full((_R * _H1, _H2)),
            full((1, _H2)),
            full((_H2, 1)),
            full((1, 1)),
        ],
        out_specs=edge_spec(1),
        out_shape=jax.ShapeDtypeStruct((_E, 1), jnp.float32),
    )(gr, gc, s, ws, wd, vs, vd, ug, ur, b1c, w2t, b2, W3, b3)


def kernel(edge_index, relations, concs, x, W1, b1, W2, b2, W3, b3):
    idx2 = edge_index.T.astype(jnp.int32)
    gr, gc = _sc_gather(x, idx2)

    with jax.ensure_compile_time_eval():
        maskf = (jax.random.uniform(jax.random.key(42), (_E,))
                 >= 0.5).astype(jnp.float32)[None, :]
    st = jnp.concatenate(
        [maskf, concs.T, relations.astype(jnp.float32)[None, :]], axis=0)

    w1cat = W1.transpose(1, 0, 2).reshape(2 * (_D + 1), _R * _H1)
    w1a = w1cat[0:_D]
    wca = w1cat[_D:_D + 1]
    w1b = w1cat[_D + 1:2 * _D + 1]
    wcb = w1cat[2 * _D + 1:2 * _D + 2]

    kdim = _R * _H1
    zrow = jnp.zeros((1, kdim), jnp.float32)
    ones = jnp.ones((1, kdim), jnp.float32)
    ws = (w1a + w1b) * 0.5
    wd = (w1a - w1b) * 0.5
    vcs = (wca + wcb) * 0.5
    vcd = (wca - wcb) * 0.5
    vs = jnp.concatenate([zrow, vcs, vcs, zrow], axis=0)
    vd = jnp.concatenate([zrow, vcd, -vcd, zrow], axis=0)
    ug = jnp.concatenate([2.0 * ones, zrow, zrow, zrow], axis=0)
    ur = jnp.concatenate([zrow, zrow, zrow, ones], axis=0)
    w2t = jnp.tile(W2, (_R, 1))

    return _tc_mlp(gr, gc, st, ws, wd, vs, vd, ug, ur, b1.reshape(1, kdim),
                   w2t, b2.reshape(1, _H2), W3, b3.reshape(1, 1))


# speedup vs baseline: 5.6563x; 1.5974x over previous
import functools

import jax
import jax.numpy as jnp
from jax import lax
from jax.experimental import pallas as pl
from jax.experimental.pallas import tpu as pltpu
from jax.experimental.pallas import tpu_sc as plsc

_N = 50000
_E = 800000
_D = 32
_R = 4
_H1 = 32
_H2 = 16

_NC, _NS = 2, 16
_NW = _NC * _NS
_PER_W = _E // _NW
_CHUNK = 1000
_NITER = _PER_W // _CHUNK

_B = 3200


def _sc_gather(x, idx2):
    mesh = plsc.VectorSubcoreMesh(core_axis_name="c", subcore_axis_name="s")

    @functools.partial(
        pl.kernel,
        mesh=mesh,
        compiler_params=pltpu.CompilerParams(use_tc_tiling_on_sc=False),
        out_type=jax.ShapeDtypeStruct((_E, 128), jnp.float32),
        scratch_types=[
            pltpu.VMEM((_CHUNK,), jnp.int32),
            pltpu.VMEM((_CHUNK,), jnp.int32),
            pltpu.VMEM((_CHUNK, _D), jnp.float32),
            pltpu.VMEM((_CHUNK, _D), jnp.float32),
            pltpu.SemaphoreType.DMA,
            pltpu.SemaphoreType.DMA,
        ],
    )
    def gather_k(x_hbm, idx2_hbm, out_hbm,
                 idxr_v, idxc_v, rowsr_v, rowsc_v, semr, semc):
        wid = lax.axis_index("s") * _NC + lax.axis_index("c")
        base = wid * _PER_W

        def body(i, carry):
            off = base + i * _CHUNK
            pltpu.sync_copy(idx2_hbm.at[0, pl.ds(off, _CHUNK)], idxr_v)
            pltpu.sync_copy(idx2_hbm.at[1, pl.ds(off, _CHUNK)], idxc_v)
            cr = pltpu.async_copy(x_hbm.at[idxr_v], rowsr_v, semr)
            cc = pltpu.async_copy(x_hbm.at[idxc_v], rowsc_v, semc)
            cr.wait()
            cc.wait()
            pltpu.sync_copy(rowsr_v, out_hbm.at[pl.ds(off, _CHUNK), pl.ds(0, _D)])
            pltpu.sync_copy(rowsc_v, out_hbm.at[pl.ds(off, _CHUNK), pl.ds(_D, _D)])
            return carry

        lax.fori_loop(0, _NITER, body, 0)

    return gather_k(x, idx2)


def _sdot(st, v):
    return lax.dot_general(st, v, (((0,), (0,)), ((), ())),
                           preferred_element_type=jnp.float32)


def _tc_body(z_ref, s_ref, ws_ref, wd_ref, vs_ref, vd_ref, ug_ref,
             ur_ref, b1_ref, w2t_ref, b2_ref, w3_ref, b3_ref, o_ref):
    z = z_ref[...]
    st = s_ref[...]
    zlane = lax.broadcasted_iota(jnp.int32, (_B, 128), 1)
    zs = jnp.where(zlane < 2 * _D, z, 0.0)
    u = (jnp.dot(zs, ws_ref[...], preferred_element_type=jnp.float32)
         + _sdot(st, vs_ref[...]))
    v = (jnp.dot(zs, wd_ref[...], preferred_element_type=jnp.float32)
         + _sdot(st, vd_ref[...]))
    sg = _sdot(st, ug_ref[...])
    r128 = _sdot(st, ur_ref[...])
    h = (u - v) + sg * v + b1_ref[...]
    lane = lax.broadcasted_iota(jnp.int32, (_B, _R * _H1), 1)
    group = (lane // _H1).astype(jnp.float32)
    hm = jnp.where(group == r128, jnp.maximum(h, 0.0), 0.0)
    h2 = jnp.dot(hm, w2t_ref[...], preferred_element_type=jnp.float32)
    h2 = jnp.maximum(h2 + b2_ref[...], 0.0)
    o = jnp.dot(h2, w3_ref[...], preferred_element_type=jnp.float32)
    o_ref[...] = o + b3_ref[...]


def _tc_mlp(z, s, ws, wd, vs, vd, ug, ur, b1c, w2t, b2, W3, b3):
    grid = (_E // _B,)
    edge_spec = lambda d: pl.BlockSpec((_B, d), lambda i: (i, 0))
    full = lambda sh: pl.BlockSpec(sh, lambda i: (0, 0))
    return pl.pallas_call(
        _tc_body,
        grid=grid,
        in_specs=[
            edge_spec(128),
            pl.BlockSpec((4, _B), lambda i: (0, i)),
            full((128, _R * _H1)),
            full((128, _R * _H1)),
            full((4, _R * _H1)),
            full((4, _R * _H1)),
            full((4, _R * _H1)),
            full((4, _R * _H1)),
            full((1, _R * _H1)),
            full((_R * _H1, _H2)),
            full((1, _H2)),
            full((_H2, 1)),
            full((1, 1)),
        ],
        out_specs=edge_spec(1),
        out_shape=jax.ShapeDtypeStruct((_E, 1), jnp.float32),
    )(z, s, ws, wd, vs, vd, ug, ur, b1c, w2t, b2, W3, b3)


def kernel(edge_index, relations, concs, x, W1, b1, W2, b2, W3, b3):
    idx2 = edge_index.T.astype(jnp.int32)
    z = _sc_gather(x, idx2)

    with jax.ensure_compile_time_eval():
        maskf = (jax.random.uniform(jax.random.key(42), (_E,))
                 >= 0.5).astype(jnp.float32)[None, :]
    st = jnp.concatenate(
        [maskf, concs.T, relations.astype(jnp.float32)[None, :]], axis=0)

    w1cat = W1.transpose(1, 0, 2).reshape(2 * (_D + 1), _R * _H1)
    w1a = w1cat[0:_D]
    wca = w1cat[_D:_D + 1]
    w1b = w1cat[_D + 1:2 * _D + 1]
    wcb = w1cat[2 * _D + 1:2 * _D + 2]

    kdim = _R * _H1
    zrow = jnp.zeros((1, kdim), jnp.float32)
    ones = jnp.ones((1, kdim), jnp.float32)
    zpad = jnp.zeros((128 - 2 * _D, kdim), jnp.float32)
    ws2 = (w1a + w1b) * 0.5
    wd2 = (w1a - w1b) * 0.5
    ws = jnp.concatenate([ws2, ws2, zpad], axis=0)
    wd = jnp.concatenate([wd2, -wd2, zpad], axis=0)
    vcs = (wca + wcb) * 0.5
    vcd = (wca - wcb) * 0.5
    vs = jnp.concatenate([zrow, vcs, vcs, zrow], axis=0)
    vd = jnp.concatenate([zrow, vcd, -vcd, zrow], axis=0)
    ug = jnp.concatenate([2.0 * ones, zrow, zrow, zrow], axis=0)
    ur = jnp.concatenate([zrow, zrow, zrow, ones], axis=0)
    w2t = jnp.tile(W2, (_R, 1))

    return _tc_mlp(z, st, ws, wd, vs, vd, ug, ur, b1.reshape(1, kdim),
                   w2t, b2.reshape(1, _H2), W3, b3.reshape(1, 1))
